# baseline (speedup 1.0000x reference)
import functools

import jax
import jax.numpy as jnp
from jax import lax
from jax.experimental import pallas as pl
from jax.experimental.pallas import tpu as pltpu
from jax.experimental.pallas import tpu_sc as plsc

_NC = 2
_NS = 16
_LANES = 16


def _sc_pool(emb_table, idx, mask):
    B, L = idx.shape
    D = emb_table.shape[1]
    NW = _NC * _NS
    b_per_w = B // NW
    n_chunks = D // _LANES
    g0 = min(128, L - L % 8)
    g1 = L - g0
    assert g1 <= 128 and g0 % 8 == 0 and g1 % 8 == 0

    mesh = plsc.VectorSubcoreMesh(
        core_axis_name="c", subcore_axis_name="s",
        num_cores=_NC, num_subcores=_NS)

    @functools.partial(
        pl.kernel,
        out_type=jax.ShapeDtypeStruct((B, D), jnp.float32),
        mesh=mesh,
        scratch_types=[
            pltpu.VMEM((b_per_w, L), jnp.int32),
            pltpu.VMEM((b_per_w, L), jnp.float32),
            pltpu.VMEM((L, D), jnp.float32),
            pltpu.VMEM((b_per_w, D), jnp.float32),
            pltpu.SemaphoreType.DMA,
        ],
        compiler_params=pltpu.CompilerParams(use_tc_tiling_on_sc=False),
    )
    def pool_kernel(table_hbm, idx_hbm, mask_hbm, out_hbm,
                    idx_v, mask_v, rows_v, out_v, sem):
        wid = lax.axis_index("s") * _NC + lax.axis_index("c")
        base = wid * b_per_w
        pltpu.sync_copy(idx_hbm.at[pl.ds(base, b_per_w)], idx_v)
        pltpu.sync_copy(mask_hbm.at[pl.ds(base, b_per_w)], mask_v)

        def row_body(b, _):
            cp0 = pltpu.make_async_copy(
                table_hbm.at[idx_v.at[b, pl.ds(0, g0)]],
                rows_v.at[pl.ds(0, g0)], sem)
            cp1 = pltpu.make_async_copy(
                table_hbm.at[idx_v.at[b, pl.ds(g0, g1)]],
                rows_v.at[pl.ds(g0, g1)], sem)
            cp0.start()
            cp1.start()
            cp0.wait()
            cp1.wait()

            def acc_body(l, carry):
                start = jnp.minimum((l // _LANES) * _LANES, L - _LANES)
                mvec = mask_v[b, pl.ds(start, _LANES)]
                m = jnp.take_along_axis(
                    mvec, jnp.full((_LANES,), l - start, jnp.int32), axis=0,
                    mode="promise_in_bounds")
                return tuple(
                    carry[c] + rows_v[l, pl.ds(c * _LANES, _LANES)] * m
                    for c in range(n_chunks))

            acc = lax.fori_loop(
                0, L, acc_body,
                tuple(jnp.zeros((_LANES,), jnp.float32)
                      for _ in range(n_chunks)))
            for c in range(n_chunks):
                out_v[b, pl.ds(c * _LANES, _LANES)] = acc[c]
            return 0

        lax.fori_loop(0, b_per_w, row_body, 0)
        pltpu.sync_copy(out_v, out_hbm.at[pl.ds(base, b_per_w)])

    return pool_kernel(emb_table, idx, mask)


def _tc_finish(pooled, mask, W, b2d):
    B, D = pooled.shape
    L = mask.shape[1]
    H = W.shape[1]
    blk = 512

    def body(pooled_ref, mask_ref, w_ref, b_ref, out_ref):
        denom = jnp.sum(mask_ref[...], axis=1, keepdims=True) + 1e-9
        mean = pooled_ref[...] / denom
        out_ref[...] = jnp.dot(
            mean, w_ref[...], preferred_element_type=jnp.float32) + b_ref[...]

    return pl.pallas_call(
        body,
        grid=(B // blk,),
        in_specs=[
            pl.BlockSpec((blk, D), lambda i: (i, 0)),
            pl.BlockSpec((blk, L), lambda i: (i, 0)),
            pl.BlockSpec((D, H), lambda i: (0, 0)),
            pl.BlockSpec((1, H), lambda i: (0, 0)),
        ],
        out_specs=pl.BlockSpec((blk, H), lambda i: (i, 0)),
        out_shape=jax.ShapeDtypeStruct((B, H), jnp.float32),
    )(pooled, mask, W, b2d)


def kernel(mid_batch_ph, mid_his_batch_ph, mask, emb_table, W, b):
    pooled = _sc_pool(emb_table, mid_his_batch_ph, mask)
    return _tc_finish(pooled, mask, W, b.reshape(1, -1))


# speedup vs baseline: 1.1885x; 1.1885x over previous
import functools

import jax
import jax.numpy as jnp
from jax import lax
from jax.experimental import pallas as pl
from jax.experimental.pallas import tpu as pltpu
from jax.experimental.pallas import tpu_sc as plsc

_NC = 2
_NS = 16
_LANES = 16


def _sc_pool(emb_table, idx, mask):
    B, L = idx.shape
    D = emb_table.shape[1]
    NW = _NC * _NS
    b_per_w = B // NW
    n_chunks = D // _LANES
    g0 = min(128, L - L % 8)
    g1 = L - g0
    assert g1 <= 128 and g0 % 8 == 0 and g1 % 8 == 0
    n_groups = L // _LANES
    tail = L - n_groups * _LANES

    mesh = plsc.VectorSubcoreMesh(
        core_axis_name="c", subcore_axis_name="s",
        num_cores=_NC, num_subcores=_NS)

    @functools.partial(
        pl.kernel,
        out_type=jax.ShapeDtypeStruct((B, D), jnp.float32),
        mesh=mesh,
        scratch_types=[
            pltpu.VMEM((b_per_w, L), jnp.int32),
            pltpu.VMEM((b_per_w, L), jnp.float32),
            pltpu.VMEM((L, D), jnp.float32),
            pltpu.VMEM((L, D), jnp.float32),
            pltpu.VMEM((b_per_w, D), jnp.float32),
            pltpu.SemaphoreType.DMA,
            pltpu.SemaphoreType.DMA,
        ],
        compiler_params=pltpu.CompilerParams(use_tc_tiling_on_sc=False),
    )
    def pool_kernel(table_hbm, idx_hbm, mask_hbm, out_hbm,
                    idx_v, mask_v, rows_a, rows_b, out_v, sem_a, sem_b):
        wid = lax.axis_index("s") * _NC + lax.axis_index("c")
        base = wid * b_per_w
        pltpu.sync_copy(idx_hbm.at[pl.ds(base, b_per_w)], idx_v)
        pltpu.sync_copy(mask_hbm.at[pl.ds(base, b_per_w)], mask_v)

        def copies(b, rows_v, sem):
            return (
                pltpu.make_async_copy(
                    table_hbm.at[idx_v.at[b, pl.ds(0, g0)]],
                    rows_v.at[pl.ds(0, g0)], sem),
                pltpu.make_async_copy(
                    table_hbm.at[idx_v.at[b, pl.ds(g0, g1)]],
                    rows_v.at[pl.ds(g0, g1)], sem),
            )

        def gather_start(b, rows_v, sem):
            for cp in copies(b, rows_v, sem):
                cp.start()

        def gather_wait(b, rows_v, sem):
            for cp in copies(b, rows_v, sem):
                cp.wait()

        zero16 = lax.iota(jnp.int32, _LANES) * 0

        def splat(mvec, j):
            return jnp.take_along_axis(mvec, zero16 + j, axis=0,
                                       mode="promise_in_bounds")

        def acc_row(b, rows_v):
            def group_body(g, carry):
                mvec = mask_v[b, pl.ds(g * _LANES, _LANES)]
                accs = list(carry)
                for j in range(_LANES):
                    m = splat(mvec, j)
                    l = g * _LANES + j
                    for c in range(n_chunks):
                        accs[c] = accs[c] + (
                            rows_v[l, pl.ds(c * _LANES, _LANES)] * m)
                return tuple(accs)

            accs = lax.fori_loop(
                0, n_groups, group_body,
                tuple(jnp.zeros((_LANES,), jnp.float32)
                      for _ in range(n_chunks)))
            if tail:
                mvec = mask_v[b, pl.ds(L - _LANES, _LANES)]
                accs = list(accs)
                for j in range(_LANES - tail, _LANES):
                    m = splat(mvec, j)
                    l = L - _LANES + j
                    for c in range(n_chunks):
                        accs[c] = accs[c] + (
                            rows_v[l, pl.ds(c * _LANES, _LANES)] * m)
            for c in range(n_chunks):
                out_v[b, pl.ds(c * _LANES, _LANES)] = accs[c]

        gather_start(0, rows_a, sem_a)

        def pair_body(i, _):
            b0 = 2 * i
            b1 = b0 + 1
            gather_start(b1, rows_b, sem_b)
            gather_wait(b0, rows_a, sem_a)
            acc_row(b0, rows_a)
            gather_start(jnp.minimum(b1 + 1, b_per_w - 1), rows_a, sem_a)
            gather_wait(b1, rows_b, sem_b)
            acc_row(b1, rows_b)
            return 0

        lax.fori_loop(0, b_per_w // 2, pair_body, 0)
        gather_wait(0, rows_a, sem_a)
        pltpu.sync_copy(out_v, out_hbm.at[pl.ds(base, b_per_w)])

    return pool_kernel(emb_table, idx, mask)


def _tc_finish(pooled, mask, W, b2d):
    B, D = pooled.shape
    L = mask.shape[1]
    H = W.shape[1]
    blk = 512

    def body(pooled_ref, mask_ref, w_ref, b_ref, out_ref):
        denom = jnp.sum(mask_ref[...], axis=1, keepdims=True) + 1e-9
        mean = pooled_ref[...] / denom
        out_ref[...] = jnp.dot(
            mean, w_ref[...], preferred_element_type=jnp.float32) + b_ref[...]

    return pl.pallas_call(
        body,
        grid=(B // blk,),
        in_specs=[
            pl.BlockSpec((blk, D), lambda i: (i, 0)),
            pl.BlockSpec((blk, L), lambda i: (i, 0)),
            pl.BlockSpec((D, H), lambda i: (0, 0)),
            pl.BlockSpec((1, H), lambda i: (0, 0)),
        ],
        out_specs=pl.BlockSpec((blk, H), lambda i: (i, 0)),
        out_shape=jax.ShapeDtypeStruct((B, H), jnp.float32),
    )(pooled, mask, W, b2d)


def kernel(mid_batch_ph, mid_his_batch_ph, mask, emb_table, W, b):
    pooled = _sc_pool(emb_table, mid_his_batch_ph, mask)
    return _tc_finish(pooled, mask, W, b.reshape(1, -1))
